# baseline (speedup 1.0000x reference)
import functools
import jax
import jax.numpy as jnp
from jax import lax
from jax.experimental import pallas as pl
from jax.experimental.pallas import tpu as pltpu
from jax.experimental.pallas import tpu_sc as plsc

_C = 5
_FILL = -999.0
_NC = 1
_NS = 16
_L = 16


def _make_sc_vals(B, Lx, Lc, Lt, D):
    n_workers = _NC * _NS
    assert n_workers % B == 0
    per_batch = n_workers // B
    chunk = Lc // per_batch
    assert chunk % _L == 0

    mesh = plsc.VectorSubcoreMesh(core_axis_name="c", subcore_axis_name="s",
                                  num_cores=_NC, num_subcores=_NS)

    @functools.partial(
        pl.kernel,
        out_type=jax.ShapeDtypeStruct((B, Lc), jnp.float32),
        mesh=mesh,
        scratch_types=[
            pltpu.VMEM((B,), jnp.int32),
            pltpu.VMEM((B,), jnp.int32),
            pltpu.VMEM((_L,), jnp.float32),
            pltpu.VMEM((_L,), jnp.float32),
            pltpu.VMEM((Lt,), jnp.float32),
            pltpu.VMEM((chunk,), jnp.float32),
            pltpu.SemaphoreType.DMA,
            pltpu.SemaphoreType.DMA,
        ],
    )
    def sc_vals(time2d_hbm, x_hbm, lenx_hbm, lenc_hbm, vals_hbm,
                lenx_v, lenc_v, x0_v, xl_v, row_v, out_v, sem, sem2):
        wid = lax.axis_index("s") * _NC + lax.axis_index("c")
        i = wid // per_batch
        j0 = (wid % per_batch) * chunk

        c_lx = pltpu.async_copy(lenx_hbm, lenx_v, sem)
        c_lc = pltpu.async_copy(lenc_hbm, lenc_v, sem)
        c_row = pltpu.async_copy(time2d_hbm.at[i], row_v, sem2)
        c_x0 = pltpu.async_copy(x_hbm.at[i, 0, pl.ds(0, _L)], x0_v, sem2)
        c_lx.wait()
        c_lc.wait()
        lane = lax.iota(jnp.int32, _L)
        lx = lenx_v[pl.ds(i, 1)][0]
        lc = lenc_v[pl.ds(i, 1)][0]
        c_xl = pltpu.async_copy(x_hbm.at[i, lx - 1, pl.ds(0, _L)], xl_v, sem)
        c_xl.wait()
        c_x0.wait()
        c_row.wait()
        x0 = x0_v[pl.ds(_C, 1)][0]
        xl = xl_v[pl.ds(_C, 1)][0]
        t0 = row_v[pl.ds(0, 1)][0]
        tl = row_v[pl.ds(lx - 1, 1)][0]
        x0v = jnp.full((_L,), x0, jnp.float32)
        t0v = jnp.full((_L,), t0, jnp.float32)
        slope = (jnp.full((_L,), xl, jnp.float32) - x0v) / (
            jnp.full((_L,), tl, jnp.float32) - t0v)

        def step(it, carry):
            base = it * (2 * _L)
            for u in range(2):
                off = base + u * _L
                t = row_v[pl.ds(lx + j0 + off, _L)]
                val = slope * (t - t0v) + x0v
                jv = j0 + off + lane
                val = jnp.where(jv < lc, val, _FILL)
                out_v[pl.ds(off, _L)] = val
            return carry

        lax.fori_loop(0, chunk // (2 * _L), step, 0)
        pltpu.sync_copy(out_v, vals_hbm.at[i, pl.ds(j0, chunk)])

    return sc_vals


def _tc_fill_body(vals_ref, out_ref):
    nb, Lc, D = out_ref.shape
    lane = lax.broadcasted_iota(jnp.int32, (Lc, D), 1)
    for b in range(nb):
        v = vals_ref[b].reshape(Lc, 1)
        out_ref[b] = jnp.where(lane == _C, v, _FILL)


def kernel(x, time, context, len_x, len_context, len_time):
    B, Lx, D = x.shape
    Lc = context.shape[1]
    Lt = time.shape[1]

    sc_vals = _make_sc_vals(B, Lx, Lc, Lt, D)
    vals = sc_vals(time, x,
                   len_x.astype(jnp.int32), len_context.astype(jnp.int32))

    nb = 8
    out = pl.pallas_call(
        _tc_fill_body,
        grid=(B // nb,),
        in_specs=[pl.BlockSpec((nb, Lc), lambda i: (i, 0))],
        out_specs=pl.BlockSpec((nb, Lc, D), lambda i: (i, 0, 0)),
        out_shape=jax.ShapeDtypeStruct((B, Lc, D), x.dtype),
    )(vals)
    return out


# speedup vs baseline: 1.0268x; 1.0268x over previous
import functools
import jax
import jax.numpy as jnp
from jax import lax
from jax.experimental import pallas as pl
from jax.experimental.pallas import tpu as pltpu
from jax.experimental.pallas import tpu_sc as plsc

_C = 5
_FILL = -999.0
_NC = 1
_NS = 16
_L = 16


def _make_sc_vals(B, Lx, Lc, Lt, D):
    n_workers = _NC * _NS
    assert n_workers % B == 0
    per_batch = n_workers // B
    chunk = Lc // per_batch
    assert chunk % _L == 0

    mesh = plsc.VectorSubcoreMesh(core_axis_name="c", subcore_axis_name="s",
                                  num_cores=_NC, num_subcores=_NS)

    @functools.partial(
        pl.kernel,
        out_type=jax.ShapeDtypeStruct((B, Lc), jnp.float32),
        mesh=mesh,
        scratch_types=[
            pltpu.VMEM((B,), jnp.int32),
            pltpu.VMEM((B,), jnp.int32),
            pltpu.VMEM((_L,), jnp.float32),
            pltpu.VMEM((_L,), jnp.float32),
            pltpu.VMEM((Lt,), jnp.float32),
            pltpu.VMEM((chunk,), jnp.float32),
            pltpu.SemaphoreType.DMA,
            pltpu.SemaphoreType.DMA,
        ],
    )
    def sc_vals(time2d_hbm, x_hbm, lenx_hbm, lenc_hbm, vals_hbm,
                lenx_v, lenc_v, x0_v, xl_v, row_v, out_v, sem, sem2):
        wid = lax.axis_index("s") * _NC + lax.axis_index("c")
        i = wid // per_batch
        j0 = (wid % per_batch) * chunk

        c_lx = pltpu.async_copy(lenx_hbm, lenx_v, sem)
        c_lc = pltpu.async_copy(lenc_hbm, lenc_v, sem)
        c_row = pltpu.async_copy(time2d_hbm.at[i], row_v, sem2)
        c_x0 = pltpu.async_copy(x_hbm.at[i, 0, pl.ds(0, _L)], x0_v, sem2)
        c_lx.wait()
        c_lc.wait()
        lane = lax.iota(jnp.int32, _L)
        lx = lenx_v[pl.ds(i, 1)][0]
        lc = lenc_v[pl.ds(i, 1)][0]
        c_xl = pltpu.async_copy(x_hbm.at[i, lx - 1, pl.ds(0, _L)], xl_v, sem)
        c_xl.wait()
        c_x0.wait()
        c_row.wait()
        x0 = x0_v[pl.ds(_C, 1)][0]
        xl = xl_v[pl.ds(_C, 1)][0]
        t0 = row_v[pl.ds(0, 1)][0]
        tl = row_v[pl.ds(lx - 1, 1)][0]
        x0v = jnp.full((_L,), x0, jnp.float32)
        t0v = jnp.full((_L,), t0, jnp.float32)
        slope = (jnp.full((_L,), xl, jnp.float32) - x0v) / (
            jnp.full((_L,), tl, jnp.float32) - t0v)

        def step(it, carry):
            base = it * (2 * _L)
            for u in range(2):
                off = base + u * _L
                t = row_v[pl.ds(lx + j0 + off, _L)]
                val = slope * (t - t0v) + x0v
                jv = j0 + off + lane
                val = jnp.where(jv < lc, val, _FILL)
                out_v[pl.ds(off, _L)] = val
            return carry

        lax.fori_loop(0, chunk // (2 * _L), step, 0)
        pltpu.sync_copy(out_v, vals_hbm.at[i, pl.ds(j0, chunk)])

    return sc_vals


def _tc_fill_body(vals_ref, out_ref):
    nb, Lc, D = out_ref.shape
    lane = lax.broadcasted_iota(jnp.int32, (Lc, D), 1)
    for b in range(nb):
        v = vals_ref[b].reshape(Lc, 1)
        out_ref[b] = jnp.where(lane == _C, v, _FILL)


def kernel(x, time, context, len_x, len_context, len_time):
    B, Lx, D = x.shape
    Lc = context.shape[1]
    Lt = time.shape[1]

    sc_vals = _make_sc_vals(B, Lx, Lc, Lt, D)
    vals = sc_vals(time, x,
                   len_x.astype(jnp.int32), len_context.astype(jnp.int32))

    nb, nj = 8, 2
    out = pl.pallas_call(
        _tc_fill_body,
        grid=(B // nb, nj),
        in_specs=[pl.BlockSpec((nb, Lc // nj), lambda i, j: (i, j))],
        out_specs=pl.BlockSpec((nb, Lc // nj, D), lambda i, j: (i, j, 0)),
        out_shape=jax.ShapeDtypeStruct((B, Lc, D), x.dtype),
    )(vals)
    return out
